# baseline (speedup 1.0000x reference)
import functools

import jax
import jax.numpy as jnp
from jax import lax
from jax.experimental import pallas as pl
from jax.experimental.pallas import tpu as pltpu
from jax.experimental.pallas import tpu_sc as plsc

_BATCH = 16384
_BIT = 64
_NC = 2
_NS = 16
_NW = _NC * _NS
_BPW = _BATCH // _NW
_L = 16


@functools.partial(
    pl.kernel,
    out_type=jax.ShapeDtypeStruct((_NW, _L), jnp.float32),
    mesh=plsc.VectorSubcoreMesh(core_axis_name="c", subcore_axis_name="s"),
    compiler_params=pltpu.CompilerParams(use_tc_tiling_on_sc=False),
    scratch_types=[
        pltpu.VMEM((_BPW,), jnp.int32),
        pltpu.VMEM((_BPW, _BIT), jnp.float32),
        pltpu.VMEM((_BPW, _BIT), jnp.float32),
        pltpu.VMEM((_L,), jnp.float32),
        pltpu.SemaphoreType.DMA,
        pltpu.SemaphoreType.DMA,
    ],
)
def _sc_sq_err(u_hbm, ind_hbm, h_hbm, out_hbm, idx_v, u_v, g_v, part_v,
               sem_u, sem_g):
    wid = lax.axis_index("s") * _NC + lax.axis_index("c")
    base = wid * _BPW

    pltpu.sync_copy(ind_hbm.at[pl.ds(base, _BPW)], idx_v)
    cp_u = pltpu.async_copy(u_hbm.at[pl.ds(base, _BPW)], u_v, sem_u)
    cp_g = pltpu.async_copy(h_hbm.at[idx_v], g_v, sem_g)
    cp_u.wait()
    cp_g.wait()

    zero = jnp.zeros((_L,), jnp.float32)

    def body(i, accs):
        a0, a1, a2, a3 = accs
        d0 = u_v[i, pl.ds(0, _L)] - g_v[i, pl.ds(0, _L)]
        d1 = u_v[i, pl.ds(_L, _L)] - g_v[i, pl.ds(_L, _L)]
        d2 = u_v[i, pl.ds(2 * _L, _L)] - g_v[i, pl.ds(2 * _L, _L)]
        d3 = u_v[i, pl.ds(3 * _L, _L)] - g_v[i, pl.ds(3 * _L, _L)]
        return (a0 + d0 * d0, a1 + d1 * d1, a2 + d2 * d2, a3 + d3 * d3)

    a0, a1, a2, a3 = lax.fori_loop(0, _BPW, body, (zero, zero, zero, zero))
    part_v[...] = (a0 + a1) + (a2 + a3)
    pltpu.sync_copy(part_v, out_hbm.at[wid])


def kernel(u, ind, H):
    partials = _sc_sq_err(u, ind, H)
    return jnp.sum(partials) / jnp.float32(_BATCH * _BIT)


# speedup vs baseline: 1.3103x; 1.3103x over previous
import functools

import jax
import jax.numpy as jnp
from jax import lax
from jax.experimental import pallas as pl
from jax.experimental.pallas import tpu as pltpu
from jax.experimental.pallas import tpu_sc as plsc

_BATCH = 16384
_BIT = 64
_NC = 2
_NS = 16
_NW = _NC * _NS
_BPW = _BATCH // _NW
_CHUNK = 128
_NCHUNK = _BPW // _CHUNK
_L = 16


@functools.partial(
    pl.kernel,
    out_type=jax.ShapeDtypeStruct((_NW, _L), jnp.float32),
    mesh=plsc.VectorSubcoreMesh(core_axis_name="c", subcore_axis_name="s"),
    scratch_types=[
        pltpu.VMEM((_BPW,), jnp.int32),
        pltpu.VMEM((2, _CHUNK, _BIT), jnp.float32),
        pltpu.VMEM((2, _CHUNK, _BIT), jnp.float32),
        pltpu.VMEM((_L,), jnp.float32),
        pltpu.SemaphoreType.DMA,
        pltpu.SemaphoreType.DMA,
        pltpu.SemaphoreType.DMA,
        pltpu.SemaphoreType.DMA,
    ],
)
def _sc_sq_err(u_hbm, ind_hbm, h_hbm, out_hbm, idx_v, u_v, g_v, part_v,
               sem_u0, sem_u1, sem_g0, sem_g1):
    wid = lax.axis_index("s") * _NC + lax.axis_index("c")
    base = wid * _BPW

    pltpu.sync_copy(ind_hbm.at[pl.ds(base, _BPW)], idx_v)

    sem_u = (sem_u0, sem_u1)
    sem_g = (sem_g0, sem_g1)

    def start_chunk(c, slot):
        pltpu.async_copy(
            u_hbm.at[pl.ds(base + c * _CHUNK, _CHUNK)], u_v.at[slot],
            sem_u[slot])

        def issue(jj, _):
            vec = idx_v[pl.ds(c * _CHUNK + jj * _L, _L)]
            for k in range(_L):
                pltpu.async_copy(h_hbm.at[vec[k]], g_v.at[slot, jj * _L + k],
                                 sem_g[slot])
            return 0

        lax.fori_loop(0, _CHUNK // _L, issue, 0)

    def wait_chunk(slot):
        pltpu.make_async_copy(
            u_hbm.at[pl.ds(0, _CHUNK)], u_v.at[slot], sem_u[slot]).wait()

        def drain(j, _):
            pltpu.make_async_copy(
                h_hbm.at[0], g_v.at[slot, 0], sem_g[slot]).wait()
            return 0

        lax.fori_loop(0, _CHUNK, drain, 0)

    zero = jnp.zeros((_L,), jnp.float32)

    def accum(slot, accs):
        def body(i, accs):
            a0, a1, a2, a3 = accs
            d0 = u_v[slot, i, pl.ds(0, _L)] - g_v[slot, i, pl.ds(0, _L)]
            d1 = u_v[slot, i, pl.ds(_L, _L)] - g_v[slot, i, pl.ds(_L, _L)]
            d2 = (u_v[slot, i, pl.ds(2 * _L, _L)]
                  - g_v[slot, i, pl.ds(2 * _L, _L)])
            d3 = (u_v[slot, i, pl.ds(3 * _L, _L)]
                  - g_v[slot, i, pl.ds(3 * _L, _L)])
            return (a0 + d0 * d0, a1 + d1 * d1, a2 + d2 * d2, a3 + d3 * d3)

        return lax.fori_loop(0, _CHUNK, body, accs)

    start_chunk(0, 0)
    accs = (zero, zero, zero, zero)
    for c in range(_NCHUNK):
        slot = c % 2
        if c + 1 < _NCHUNK:
            start_chunk(c + 1, 1 - slot)
        wait_chunk(slot)
        accs = accum(slot, accs)

    a0, a1, a2, a3 = accs
    part_v[...] = (a0 + a1) + (a2 + a3)
    pltpu.sync_copy(part_v, out_hbm.at[wid])


def kernel(u, ind, H):
    partials = _sc_sq_err(u, ind, H)
    return jnp.sum(partials) / jnp.float32(_BATCH * _BIT)


# speedup vs baseline: 1.3123x; 1.0016x over previous
import functools

import jax
import jax.numpy as jnp
from jax import lax
from jax.experimental import pallas as pl
from jax.experimental.pallas import tpu as pltpu
from jax.experimental.pallas import tpu_sc as plsc

_BATCH = 16384
_BIT = 64
_NC = 2
_NS = 16
_NW = _NC * _NS
_BPW = _BATCH // _NW
_CHUNK = 128
_NCHUNK = _BPW // _CHUNK
_L = 16


@functools.partial(
    pl.kernel,
    out_type=jax.ShapeDtypeStruct((_NW, _L), jnp.float32),
    mesh=plsc.VectorSubcoreMesh(core_axis_name="c", subcore_axis_name="s"),
    compiler_params=pltpu.CompilerParams(
        skip_device_barrier=True,
        disable_bounds_checks=True,
        disable_semaphore_checks=True,
    ),
    scratch_types=[
        pltpu.VMEM((_BPW,), jnp.int32),
        pltpu.VMEM((2, _CHUNK, _BIT), jnp.float32),
        pltpu.VMEM((2, _CHUNK, _BIT), jnp.float32),
        pltpu.VMEM((_L,), jnp.float32),
        pltpu.SemaphoreType.DMA,
        pltpu.SemaphoreType.DMA,
        pltpu.SemaphoreType.DMA,
        pltpu.SemaphoreType.DMA,
    ],
)
def _sc_sq_err(u_hbm, ind_hbm, h_hbm, out_hbm, idx_v, u_v, g_v, part_v,
               sem_u0, sem_u1, sem_g0, sem_g1):
    wid = lax.axis_index("s") * _NC + lax.axis_index("c")
    base = wid * _BPW

    pltpu.sync_copy(ind_hbm.at[pl.ds(base, _BPW)], idx_v)

    sem_u = (sem_u0, sem_u1)
    sem_g = (sem_g0, sem_g1)

    def start_chunk(c, slot):
        pltpu.async_copy(
            u_hbm.at[pl.ds(base + c * _CHUNK, _CHUNK)], u_v.at[slot],
            sem_u[slot])

        def issue(jj, _):
            vec = idx_v[pl.ds(c * _CHUNK + jj * _L, _L)]
            for k in range(_L):
                pltpu.async_copy(h_hbm.at[vec[k]], g_v.at[slot, jj * _L + k],
                                 sem_g[slot])
            return 0

        lax.fori_loop(0, _CHUNK // _L, issue, 0)

    def wait_chunk(slot):
        pltpu.make_async_copy(
            u_hbm.at[pl.ds(0, _CHUNK)], u_v.at[slot], sem_u[slot]).wait()

        def drain(j, _):
            pltpu.make_async_copy(
                h_hbm.at[0], g_v.at[slot, 0], sem_g[slot]).wait()
            return 0

        lax.fori_loop(0, _CHUNK, drain, 0)

    zero = jnp.zeros((_L,), jnp.float32)

    def accum(slot, accs):
        def body(i, accs):
            a0, a1, a2, a3 = accs
            d0 = u_v[slot, i, pl.ds(0, _L)] - g_v[slot, i, pl.ds(0, _L)]
            d1 = u_v[slot, i, pl.ds(_L, _L)] - g_v[slot, i, pl.ds(_L, _L)]
            d2 = (u_v[slot, i, pl.ds(2 * _L, _L)]
                  - g_v[slot, i, pl.ds(2 * _L, _L)])
            d3 = (u_v[slot, i, pl.ds(3 * _L, _L)]
                  - g_v[slot, i, pl.ds(3 * _L, _L)])
            return (a0 + d0 * d0, a1 + d1 * d1, a2 + d2 * d2, a3 + d3 * d3)

        return lax.fori_loop(0, _CHUNK, body, accs)

    start_chunk(0, 0)
    accs = (zero, zero, zero, zero)
    for c in range(_NCHUNK):
        slot = c % 2
        if c + 1 < _NCHUNK:
            start_chunk(c + 1, 1 - slot)
        wait_chunk(slot)
        accs = accum(slot, accs)

    a0, a1, a2, a3 = accs
    part_v[...] = (a0 + a1) + (a2 + a3)
    pltpu.sync_copy(part_v, out_hbm.at[wid])


def kernel(u, ind, H):
    partials = _sc_sq_err(u, ind, H)
    return jnp.sum(partials) / jnp.float32(_BATCH * _BIT)


# speedup vs baseline: 1.9916x; 1.5176x over previous
import functools

import jax
import jax.numpy as jnp
from jax import lax
from jax.experimental import pallas as pl
from jax.experimental.pallas import tpu as pltpu
from jax.experimental.pallas import tpu_sc as plsc

_BATCH = 16384
_BIT = 64
_NTRAIN = 100000
_NC = 2
_NS = 16
_NW = _NC * _NS
_CPW = _BIT // _NW
_CHUNK = 2048
_NCHUNK = _BATCH // _CHUNK
_L = 16


@functools.partial(
    pl.kernel,
    out_type=jax.ShapeDtypeStruct((_NW, _L), jnp.float32),
    mesh=plsc.VectorSubcoreMesh(core_axis_name="c", subcore_axis_name="s"),
    compiler_params=pltpu.CompilerParams(needs_layout_passes=False),
    scratch_types=[
        pltpu.VMEM((_NTRAIN,), jnp.float32),
        pltpu.VMEM((2, _CHUNK), jnp.int32),
        pltpu.VMEM((2, _CHUNK), jnp.float32),
        pltpu.VMEM((_L,), jnp.float32),
        pltpu.SemaphoreType.DMA,
        pltpu.SemaphoreType.DMA,
        pltpu.SemaphoreType.DMA,
        pltpu.SemaphoreType.DMA,
        pltpu.SemaphoreType.DMA,
    ],
)
def _sc_sq_err(ut_hbm, ind_hbm, ht_hbm, out_hbm, tab_v, idx_v, u_v, part_v,
               sem_t, sem_i0, sem_i1, sem_u0, sem_u1):
    wid = lax.axis_index("s") * _NC + lax.axis_index("c")

    sem_i = (sem_i0, sem_i1)
    sem_u = (sem_u0, sem_u1)

    def start_chunk(col, k, slot):
        pltpu.async_copy(
            ind_hbm.at[pl.ds(k * _CHUNK, _CHUNK)], idx_v.at[slot],
            sem_i[slot])
        pltpu.async_copy(
            ut_hbm.at[col, pl.ds(k * _CHUNK, _CHUNK)], u_v.at[slot],
            sem_u[slot])

    def wait_chunk(slot):
        pltpu.make_async_copy(
            ind_hbm.at[pl.ds(0, _CHUNK)], idx_v.at[slot], sem_i[slot]).wait()
        pltpu.make_async_copy(
            ind_hbm.at[pl.ds(0, _CHUNK)], u_v.at[slot], sem_u[slot]).wait()

    def accum(slot, accs):
        def body(j, accs):
            a0, a1 = accs
            i0 = idx_v[slot, pl.ds(2 * _L * j, _L)]
            i1 = idx_v[slot, pl.ds(2 * _L * j + _L, _L)]
            h0 = plsc.load_gather(tab_v, [i0])
            h1 = plsc.load_gather(tab_v, [i1])
            d0 = u_v[slot, pl.ds(2 * _L * j, _L)] - h0
            d1 = u_v[slot, pl.ds(2 * _L * j + _L, _L)] - h1
            return (a0 + d0 * d0, a1 + d1 * d1)

        return lax.fori_loop(0, _CHUNK // (2 * _L), body, accs)

    zero = jnp.zeros((_L,), jnp.float32)
    accs = (zero, zero)
    for cc in range(_CPW):
        col = wid * _CPW + cc
        tab = pltpu.async_copy(ht_hbm.at[col], tab_v, sem_t)
        start_chunk(col, 0, 0)
        tab.wait()
        for k in range(_NCHUNK):
            slot = k % 2
            if k + 1 < _NCHUNK:
                start_chunk(col, k + 1, 1 - slot)
            wait_chunk(slot)
            accs = accum(slot, accs)

    a0, a1 = accs
    part_v[...] = a0 + a1
    pltpu.sync_copy(part_v, out_hbm.at[wid])


def kernel(u, ind, H):
    partials = _sc_sq_err(u.T, ind, H.T)
    return jnp.sum(partials) / jnp.float32(_BATCH * _BIT)


# speedup vs baseline: 2.2326x; 1.1210x over previous
import functools

import jax
import jax.numpy as jnp
from jax import lax
from jax.experimental import pallas as pl
from jax.experimental.pallas import tpu as pltpu
from jax.experimental.pallas import tpu_sc as plsc

_BATCH = 16384
_BIT = 64
_NTRAIN = 100000
_NC = 2
_NS = 16
_NW = _NC * _NS
_CPW = _BIT // _NW
_CHUNK = 4096
_NCHUNK = _BATCH // _CHUNK
_L = 16


@functools.partial(
    pl.kernel,
    out_type=jax.ShapeDtypeStruct((_NW, _L), jnp.float32),
    mesh=plsc.VectorSubcoreMesh(core_axis_name="c", subcore_axis_name="s"),
    compiler_params=pltpu.CompilerParams(needs_layout_passes=False),
    scratch_types=[
        pltpu.VMEM((_NTRAIN,), jnp.float32),
        pltpu.VMEM((_BATCH,), jnp.int32),
        pltpu.VMEM((2, _CHUNK), jnp.float32),
        pltpu.VMEM((_L,), jnp.float32),
        pltpu.SemaphoreType.DMA,
        pltpu.SemaphoreType.DMA,
        pltpu.SemaphoreType.DMA,
        pltpu.SemaphoreType.DMA,
    ],
)
def _sc_sq_err(ut_hbm, ind_hbm, ht_hbm, out_hbm, tab_v, idx_v, u_v, part_v,
               sem_t, sem_i, sem_u0, sem_u1):
    wid = lax.axis_index("s") * _NC + lax.axis_index("c")

    sem_u = (sem_u0, sem_u1)

    cp_i = pltpu.async_copy(ind_hbm, idx_v, sem_i)

    def start_u(col, k, slot):
        pltpu.async_copy(
            ut_hbm.at[col, pl.ds(k * _CHUNK, _CHUNK)], u_v.at[slot],
            sem_u[slot])

    def wait_u(slot):
        pltpu.make_async_copy(
            ind_hbm.at[pl.ds(0, _CHUNK)], u_v.at[slot], sem_u[slot]).wait()

    def accum(base, slot, accs):
        def body(j, accs):
            a0, a1 = accs
            i0 = idx_v[pl.ds(base + 2 * _L * j, _L)]
            i1 = idx_v[pl.ds(base + 2 * _L * j + _L, _L)]
            h0 = plsc.load_gather(tab_v, [i0])
            h1 = plsc.load_gather(tab_v, [i1])
            d0 = u_v[slot, pl.ds(2 * _L * j, _L)] - h0
            d1 = u_v[slot, pl.ds(2 * _L * j + _L, _L)] - h1
            return (a0 + d0 * d0, a1 + d1 * d1)

        return lax.fori_loop(0, _CHUNK // (2 * _L), body, accs)

    zero = jnp.zeros((_L,), jnp.float32)
    accs = (zero, zero)
    first = True
    for cc in range(_CPW):
        col = wid * _CPW + cc
        tab = pltpu.async_copy(ht_hbm.at[col], tab_v, sem_t)
        start_u(col, 0, 0)
        if first:
            cp_i.wait()
            first = False
        tab.wait()
        for k in range(_NCHUNK):
            slot = k % 2
            if k + 1 < _NCHUNK:
                start_u(col, k + 1, 1 - slot)
            wait_u(slot)
            accs = accum(k * _CHUNK, slot, accs)

    a0, a1 = accs
    part_v[...] = a0 + a1
    pltpu.sync_copy(part_v, out_hbm.at[wid])


def kernel(u, ind, H):
    partials = _sc_sq_err(u.T, ind, H.T)
    return jnp.sum(partials) / jnp.float32(_BATCH * _BIT)


# speedup vs baseline: 2.2362x; 1.0016x over previous
import functools

import jax
import jax.numpy as jnp
from jax import lax
from jax.experimental import pallas as pl
from jax.experimental.pallas import tpu as pltpu
from jax.experimental.pallas import tpu_sc as plsc

_BATCH = 16384
_BIT = 64
_NTRAIN = 100000
_NC = 2
_NS = 16
_NW = _NC * _NS
_CPW = _BIT // _NW
_CHUNK = 4096
_NCHUNK = _BATCH // _CHUNK
_L = 16


@functools.partial(
    pl.kernel,
    out_type=jax.ShapeDtypeStruct((_NW, _L), jnp.float32),
    mesh=plsc.VectorSubcoreMesh(core_axis_name="c", subcore_axis_name="s"),
    compiler_params=pltpu.CompilerParams(
        needs_layout_passes=False,
        skip_device_barrier=True,
        disable_bounds_checks=True,
        disable_semaphore_checks=True,
    ),
    scratch_types=[
        pltpu.VMEM((_NTRAIN,), jnp.float32),
        pltpu.VMEM((_BATCH,), jnp.int32),
        pltpu.VMEM((2, _CHUNK), jnp.float32),
        pltpu.VMEM((_L,), jnp.float32),
        pltpu.SemaphoreType.DMA,
        pltpu.SemaphoreType.DMA,
        pltpu.SemaphoreType.DMA,
        pltpu.SemaphoreType.DMA,
    ],
)
def _sc_sq_err(ut_hbm, ind_hbm, ht_hbm, out_hbm, tab_v, idx_v, u_v, part_v,
               sem_t, sem_i, sem_u0, sem_u1):
    wid = lax.axis_index("s") * _NC + lax.axis_index("c")

    sem_u = (sem_u0, sem_u1)

    cp_i = pltpu.async_copy(ind_hbm, idx_v, sem_i)

    def start_u(col, k, slot):
        pltpu.async_copy(
            ut_hbm.at[col, pl.ds(k * _CHUNK, _CHUNK)], u_v.at[slot],
            sem_u[slot])

    def wait_u(slot):
        pltpu.make_async_copy(
            ind_hbm.at[pl.ds(0, _CHUNK)], u_v.at[slot], sem_u[slot]).wait()

    def accum(base, slot, accs):
        def body(j, accs):
            a0, a1 = accs
            i0 = idx_v[pl.ds(base + 2 * _L * j, _L)]
            i1 = idx_v[pl.ds(base + 2 * _L * j + _L, _L)]
            h0 = plsc.load_gather(tab_v, [i0])
            h1 = plsc.load_gather(tab_v, [i1])
            d0 = u_v[slot, pl.ds(2 * _L * j, _L)] - h0
            d1 = u_v[slot, pl.ds(2 * _L * j + _L, _L)] - h1
            return (a0 + d0 * d0, a1 + d1 * d1)

        return lax.fori_loop(0, _CHUNK // (2 * _L), body, accs)

    zero = jnp.zeros((_L,), jnp.float32)
    accs = (zero, zero)
    first = True
    for cc in range(_CPW):
        col = wid * _CPW + cc
        tab = pltpu.async_copy(ht_hbm.at[col], tab_v, sem_t)
        start_u(col, 0, 0)
        if first:
            cp_i.wait()
            first = False
        tab.wait()
        for k in range(_NCHUNK):
            slot = k % 2
            if k + 1 < _NCHUNK:
                start_u(col, k + 1, 1 - slot)
            wait_u(slot)
            accs = accum(k * _CHUNK, slot, accs)

    a0, a1 = accs
    part_v[...] = a0 + a1
    pltpu.sync_copy(part_v, out_hbm.at[wid])


def kernel(u, ind, H):
    partials = _sc_sq_err(u.T, ind, H.T)
    return jnp.sum(partials) / jnp.float32(_BATCH * _BIT)


# speedup vs baseline: 2.2924x; 1.0251x over previous
import functools

import jax
import jax.numpy as jnp
from jax import lax
from jax.experimental import pallas as pl
from jax.experimental.pallas import tpu as pltpu
from jax.experimental.pallas import tpu_sc as plsc

_BATCH = 16384
_BIT = 64
_NTRAIN = 100000
_NC = 2
_NS = 16
_NW = _NC * _NS
_CPW = _BIT // _NW
_CHUNK = 4096
_NCHUNK = _BATCH // _CHUNK
_L = 16


@functools.partial(
    pl.kernel,
    out_type=jax.ShapeDtypeStruct((_NW, _L), jnp.float32),
    mesh=plsc.VectorSubcoreMesh(core_axis_name="c", subcore_axis_name="s"),
    compiler_params=pltpu.CompilerParams(needs_layout_passes=False),
    scratch_types=[
        pltpu.VMEM((_NTRAIN,), jnp.float32),
        pltpu.VMEM((_BATCH,), jnp.int32),
        pltpu.VMEM((2, _CHUNK), jnp.float32),
        pltpu.VMEM((_L,), jnp.float32),
        pltpu.SemaphoreType.DMA,
        pltpu.SemaphoreType.DMA,
        pltpu.SemaphoreType.DMA,
        pltpu.SemaphoreType.DMA,
    ],
)
def _sc_sq_err(ut_hbm, ind_hbm, ht_hbm, out_hbm, tab_v, idx_v, u_v, part_v,
               sem_t, sem_i, sem_u0, sem_u1):
    wid = lax.axis_index("s") * _NC + lax.axis_index("c")

    sem_u = (sem_u0, sem_u1)

    cp_i = pltpu.async_copy(ind_hbm, idx_v, sem_i)

    def start_u(col, k, slot):
        pltpu.async_copy(
            ut_hbm.at[col, pl.ds(k * _CHUNK, _CHUNK)], u_v.at[slot],
            sem_u[slot])

    def wait_u(slot):
        pltpu.make_async_copy(
            ind_hbm.at[pl.ds(0, _CHUNK)], u_v.at[slot], sem_u[slot]).wait()

    def accum(base, slot, accs):
        def body(j, accs):
            a0, a1 = accs
            i0 = idx_v[pl.ds(base + 2 * _L * j, _L)]
            i1 = idx_v[pl.ds(base + 2 * _L * j + _L, _L)]
            h0 = plsc.load_gather(tab_v, [i0])
            h1 = plsc.load_gather(tab_v, [i1])
            d0 = u_v[slot, pl.ds(2 * _L * j, _L)] - h0
            d1 = u_v[slot, pl.ds(2 * _L * j + _L, _L)] - h1
            return (a0 + d0 * d0, a1 + d1 * d1)

        return lax.fori_loop(0, _CHUNK // (2 * _L), body, accs)

    zero = jnp.zeros((_L,), jnp.float32)
    accs = (zero, zero)
    first = True
    for cc in range(_CPW):
        col = wid * _CPW + cc
        tab = pltpu.async_copy(ht_hbm.at[col], tab_v, sem_t)
        start_u(col, 0, 0)
        if first:
            cp_i.wait()
            first = False
        tab.wait()
        for k in range(_NCHUNK):
            slot = k % 2
            if k + 1 < _NCHUNK:
                start_u(col, k + 1, 1 - slot)
            wait_u(slot)
            accs = accum(k * _CHUNK, slot, accs)

    a0, a1 = accs
    part_v[...] = (a0 + a1) * jnp.float32(1.0 / (_BATCH * _BIT))
    pltpu.sync_copy(part_v, out_hbm.at[wid])


def kernel(u, ind, H):
    partials = _sc_sq_err(u.T, ind, H.T)
    return jnp.sum(partials)
